# baseline (speedup 1.0000x reference)
import functools

import jax
import jax.numpy as jnp
from jax import lax
from jax.experimental import pallas as pl
from jax.experimental.pallas import tpu as pltpu
from jax.experimental.pallas import tpu_sc as plsc

NC = 2
NS = 16
NW = NC * NS
CHUNK = 96
ZROWS = 125
D = 64


def _sc_agg(n_nodes, n_edges, col_split):
  workers = NS if col_split else NW
  e_per = n_edges // workers
  n_chunks = -(-e_per // CHUNK)
  e_pad = n_chunks * CHUNK
  n_groups = n_chunks // 2
  rem = n_chunks - 2 * n_groups
  assert e_per % 16 == 0 and n_groups >= 2
  rows_per_s = n_nodes // NS
  nz = rows_per_s // ZROWS
  assert rows_per_s % ZROWS == 0
  nvec = D // 16
  n_tab = (NC * n_nodes) if col_split else n_nodes

  mesh = plsc.VectorSubcoreMesh(core_axis_name="c", subcore_axis_name="s")

  @functools.partial(
      pl.kernel,
      out_type=jax.ShapeDtypeStruct((n_nodes, NC * D), jnp.float32),
      mesh=mesh,
      compiler_params=pltpu.CompilerParams(use_tc_tiling_on_sc=False),
      scratch_types=[
          pltpu.VMEM((e_pad,), jnp.int32),
          pltpu.VMEM((e_pad,), jnp.int32),
          pltpu.VMEM((e_pad,), jnp.float32),
          *([pltpu.VMEM((CHUNK,), jnp.int32)] * 2),
          *([pltpu.VMEM((CHUNK,), jnp.int32)] * 2),
          *([pltpu.VMEM((CHUNK, D), jnp.float32)] * 2),
          *([pltpu.VMEM((CHUNK, D), jnp.float32)] * 2),
          pltpu.VMEM_SHARED((n_nodes, D), jnp.float32),
          *([pltpu.SemaphoreType.DMA] * 4),
      ],
  )
  def agg(table, edge, adj, out, src_v, dst_v, adj_v,
          srcs0, srcs1, dsts0, dsts1, grow0, grow1, srow0, srow1,
          acc, gsem0, gsem1, ssem0, ssem1):
    zbuf = srow0.at[pl.ds(0, ZROWS)]
    c = lax.axis_index("c")
    s = lax.axis_index("s")
    w = s if col_split else s * NC + c
    bufs = ((srcs0, dsts0, grow0, srow0, gsem0, ssem0),
            (srcs1, dsts1, grow1, srow1, gsem1, ssem1))

    def zrow(r, _):
      for j in range(nvec):
        zbuf[r, pl.ds(16 * j, 16)] = jnp.zeros((16,), jnp.float32)
      return 0
    lax.fori_loop(0, ZROWS, zrow, 0)
    for k in range(nz):
      pltpu.sync_copy(zbuf, acc.at[pl.ds(s * rows_per_s + k * ZROWS, ZROWS)])

    pltpu.sync_copy(edge.at[0, pl.ds(w * e_per, e_per)],
                    src_v.at[pl.ds(0, e_per)])
    pltpu.sync_copy(edge.at[1, pl.ds(w * e_per, e_per)],
                    dst_v.at[pl.ds(0, e_per)])
    pltpu.sync_copy(adj.at[pl.ds(w * e_per, e_per)],
                    adj_v.at[pl.ds(0, e_per)])
    for t in range((e_pad - e_per) // 16):
      src_v[pl.ds(e_per + 16 * t, 16)] = jnp.zeros((16,), jnp.int32)
      dst_v[pl.ds(e_per + 16 * t, 16)] = jnp.zeros((16,), jnp.int32)
      adj_v[pl.ds(e_per + 16 * t, 16)] = jnp.zeros((16,), jnp.float32)
    plsc.subcore_barrier()

    def load_src(b, i):
      srcs = bufs[b][0]
      for j in range(CHUNK // 16):
        v = src_v[pl.ds(i * CHUNK + 16 * j, 16)]
        if col_split:
          v = v * 2 + c
        srcs[pl.ds(16 * j, 16)] = v

    def load_dst(b, i):
      dsts = bufs[b][1]
      for j in range(CHUNK // 16):
        dsts[pl.ds(16 * j, 16)] = dst_v[pl.ds(i * CHUNK + 16 * j, 16)]

    def scale_copy(b, i):
      gbuf = bufs[b][2]
      sbuf = bufs[b][3]
      for g in range(CHUNK // 16):
        a16 = adj_v[pl.ds(i * CHUNK + 16 * g, 16)]
        for r in range(16):
          a = a16[r]
          for j in range(nvec):
            sl = pl.ds(16 * j, 16)
            sbuf[16 * g + r, sl] = gbuf[16 * g + r, sl] * a

    def issue_gather(b):
      srcs, _, gbuf, _, gsem, _ = bufs[b]
      pltpu.async_copy(table.at[srcs], gbuf, gsem)

    def wait_gather(b):
      srcs, _, gbuf, _, gsem, _ = bufs[b]
      pltpu.make_async_copy(table.at[srcs], gbuf, gsem).wait()

    def issue_scatter(b):
      _, dsts, _, sbuf, _, ssem = bufs[b]
      pltpu.async_copy(sbuf, acc.at[dsts], ssem, add=True)

    def wait_scatter(b):
      _, dsts, _, sbuf, _, ssem = bufs[b]
      pltpu.make_async_copy(sbuf, acc.at[dsts], ssem).wait()

    for b in range(2):
      load_src(b, b)
      issue_gather(b)

    def group_body(g, _):
      i0 = 2 * g
      for b in range(2):
        wait_gather(b)

        @pl.when(g > 0)
        def _():
          wait_scatter(b)
        scale_copy(b, i0 + b)
        load_dst(b, i0 + b)
        issue_scatter(b)

        @pl.when(g < n_groups - 1)
        def _():
          load_src(b, i0 + 2 + b)
          issue_gather(b)
      return 0
    lax.fori_loop(0, n_groups, group_body, 0)

    for b in range(2):
      wait_scatter(b)

    for r in range(rem):
      i = 2 * n_groups + r
      load_src(0, i)
      pltpu.sync_copy(table.at[bufs[0][0]], bufs[0][2])
      scale_copy(0, i)
      load_dst(0, i)
      pltpu.sync_copy(bufs[0][3], acc.at[bufs[0][1]], add=True)

    plsc.subcore_barrier()

    base_rows = rows_per_s // 8 * 8
    last_rows = n_nodes - (NS - 1) * base_rows
    for cc in range(NC):
      @pl.when((c == cc) & (s < NS - 1))
      def _():
        pltpu.sync_copy(acc.at[pl.ds(s * base_rows, base_rows)],
                        out.at[pl.ds(s * base_rows, base_rows),
                               pl.ds(cc * D, D)])

      @pl.when((c == cc) & (s == NS - 1))
      def _():
        pltpu.sync_copy(acc.at[pl.ds((NS - 1) * base_rows, last_rows)],
                        out.at[pl.ds((NS - 1) * base_rows, last_rows),
                               pl.ds(cc * D, D)])

  return agg


def _matmul_tc(n, f_in, f_out, bn):
  def body(x_ref, w_ref, o_ref):
    o_ref[...] = jnp.dot(x_ref[...], w_ref[...],
                         preferred_element_type=jnp.float32)
  return pl.pallas_call(
      body,
      grid=(n // bn,),
      in_specs=[
          pl.BlockSpec((bn, f_in), lambda i: (i, 0)),
          pl.BlockSpec((f_in, f_out), lambda i: (0, 0)),
      ],
      out_specs=pl.BlockSpec((bn, f_out), lambda i: (i, 0)),
      out_shape=jax.ShapeDtypeStruct((n, f_out), jnp.float32),
  )


def _relu_matmul_tc(n, f_in, f_out, bn):
  def body(p_ref, b_ref, w_ref, o_ref):
    h = jnp.maximum(p_ref[...] + b_ref[0], 0.0)
    o_ref[...] = jnp.dot(h, w_ref[...], preferred_element_type=jnp.float32)
  return pl.pallas_call(
      body,
      grid=(n // bn,),
      in_specs=[
          pl.BlockSpec((bn, f_in), lambda i: (i, 0)),
          pl.BlockSpec((1, f_in), lambda i: (0, 0)),
          pl.BlockSpec((f_in, f_out), lambda i: (0, 0)),
      ],
      out_specs=pl.BlockSpec((bn, f_out), lambda i: (i, 0)),
      out_shape=jax.ShapeDtypeStruct((n, f_out), jnp.float32),
  )


def _logsoftmax_tc(n, d, bn):
  def body(p_ref, b_ref, o_ref):
    p = p_ref[...]
    logits = p[:, :d] + p[:, d:] + b_ref[0]
    m = jnp.max(logits, axis=1, keepdims=True)
    shifted = logits - m
    lse = jnp.log(jnp.sum(jnp.exp(shifted), axis=1, keepdims=True))
    o_ref[...] = shifted - lse
  return pl.pallas_call(
      body,
      grid=(n // bn,),
      in_specs=[
          pl.BlockSpec((bn, 2 * d), lambda i: (i, 0)),
          pl.BlockSpec((1, d), lambda i: (0, 0)),
      ],
      out_specs=pl.BlockSpec((bn, d), lambda i: (i, 0)),
      out_shape=jax.ShapeDtypeStruct((n, d), jnp.float32),
  )


@jax.jit
def kernel(x, edge_index, adj_values, W1, b1, W2, b2):
  n, f_in = x.shape
  h_dim = W1.shape[1]
  o_dim = W2.shape[1]
  e = edge_index.shape[1]

  bn = 1000
  support1 = _matmul_tc(n, f_in, h_dim, bn)(x, W1)
  tbl1 = support1.reshape(NC * n, h_dim // NC)
  agg1 = _sc_agg(n, e, True)(tbl1, edge_index, adj_values)
  support2 = _relu_matmul_tc(n, h_dim, o_dim, bn)(
      agg1, b1.reshape(1, h_dim), W2)
  part2 = _sc_agg(n, e, False)(support2, edge_index, adj_values)
  return _logsoftmax_tc(n, o_dim, bn)(part2, b2.reshape(1, o_dim))


# speedup vs baseline: 1.0014x; 1.0014x over previous
import functools

import jax
import jax.numpy as jnp
from jax import lax
from jax.experimental import pallas as pl
from jax.experimental.pallas import tpu as pltpu
from jax.experimental.pallas import tpu_sc as plsc

NC = 2
NS = 16
NW = NC * NS
CHUNK = 64
ZROWS = 125
D = 64


def _sc_agg(n_nodes, n_edges, col_split):
  workers = NS if col_split else NW
  e_per = n_edges // workers
  n_chunks = -(-e_per // CHUNK)
  e_pad = n_chunks * CHUNK
  n_groups = n_chunks // 2
  rem = n_chunks - 2 * n_groups
  assert e_per % 16 == 0 and n_groups >= 2
  rows_per_s = n_nodes // NS
  nz = rows_per_s // ZROWS
  assert rows_per_s % ZROWS == 0
  nvec = D // 16
  n_tab = (NC * n_nodes) if col_split else n_nodes

  mesh = plsc.VectorSubcoreMesh(core_axis_name="c", subcore_axis_name="s")

  @functools.partial(
      pl.kernel,
      out_type=jax.ShapeDtypeStruct((n_nodes, NC * D), jnp.float32),
      mesh=mesh,
      compiler_params=pltpu.CompilerParams(use_tc_tiling_on_sc=False),
      scratch_types=[
          pltpu.VMEM((e_pad,), jnp.int32),
          pltpu.VMEM((e_pad,), jnp.int32),
          pltpu.VMEM((e_pad,), jnp.float32),
          *([pltpu.VMEM((CHUNK,), jnp.int32)] * 2),
          *([pltpu.VMEM((CHUNK,), jnp.int32)] * 2),
          *([pltpu.VMEM((CHUNK, D), jnp.float32)] * 2),
          *([pltpu.VMEM((CHUNK, D), jnp.float32)] * 2),
          pltpu.VMEM_SHARED((n_nodes, D), jnp.float32),
          *([pltpu.SemaphoreType.DMA] * 4),
      ],
  )
  def agg(table, edge, adj, out, src_v, dst_v, adj_v,
          srcs0, srcs1, dsts0, dsts1, grow0, grow1, srow0, srow1,
          acc, gsem0, gsem1, ssem0, ssem1):
    zbuf = srow0.at[pl.ds(0, ZROWS)]
    c = lax.axis_index("c")
    s = lax.axis_index("s")
    w = s if col_split else s * NC + c
    bufs = ((srcs0, dsts0, grow0, srow0, gsem0, ssem0),
            (srcs1, dsts1, grow1, srow1, gsem1, ssem1))

    def zrow(r, _):
      for j in range(nvec):
        zbuf[r, pl.ds(16 * j, 16)] = jnp.zeros((16,), jnp.float32)
      return 0
    lax.fori_loop(0, ZROWS, zrow, 0)
    for k in range(nz):
      pltpu.sync_copy(zbuf, acc.at[pl.ds(s * rows_per_s + k * ZROWS, ZROWS)])

    pltpu.sync_copy(edge.at[0, pl.ds(w * e_per, e_per)],
                    src_v.at[pl.ds(0, e_per)])
    pltpu.sync_copy(edge.at[1, pl.ds(w * e_per, e_per)],
                    dst_v.at[pl.ds(0, e_per)])
    pltpu.sync_copy(adj.at[pl.ds(w * e_per, e_per)],
                    adj_v.at[pl.ds(0, e_per)])
    for t in range((e_pad - e_per) // 16):
      src_v[pl.ds(e_per + 16 * t, 16)] = jnp.zeros((16,), jnp.int32)
      dst_v[pl.ds(e_per + 16 * t, 16)] = jnp.zeros((16,), jnp.int32)
      adj_v[pl.ds(e_per + 16 * t, 16)] = jnp.zeros((16,), jnp.float32)
    plsc.subcore_barrier()

    def load_src(b, i):
      srcs = bufs[b][0]
      for j in range(CHUNK // 16):
        v = src_v[pl.ds(i * CHUNK + 16 * j, 16)]
        if col_split:
          v = v * 2 + c
        srcs[pl.ds(16 * j, 16)] = v

    def load_dst(b, i):
      dsts = bufs[b][1]
      for j in range(CHUNK // 16):
        dsts[pl.ds(16 * j, 16)] = dst_v[pl.ds(i * CHUNK + 16 * j, 16)]

    def scale_copy(b, i):
      gbuf = bufs[b][2]
      sbuf = bufs[b][3]
      for g in range(CHUNK // 16):
        a16 = adj_v[pl.ds(i * CHUNK + 16 * g, 16)]
        for r in range(16):
          a = a16[r]
          for j in range(nvec):
            sl = pl.ds(16 * j, 16)
            sbuf[16 * g + r, sl] = gbuf[16 * g + r, sl] * a

    def issue_gather(b):
      srcs, _, gbuf, _, gsem, _ = bufs[b]
      pltpu.async_copy(table.at[srcs], gbuf, gsem)

    def wait_gather(b):
      srcs, _, gbuf, _, gsem, _ = bufs[b]
      pltpu.make_async_copy(table.at[srcs], gbuf, gsem).wait()

    def issue_scatter(b):
      _, dsts, _, sbuf, _, ssem = bufs[b]
      pltpu.async_copy(sbuf, acc.at[dsts], ssem, add=True)

    def wait_scatter(b):
      _, dsts, _, sbuf, _, ssem = bufs[b]
      pltpu.make_async_copy(sbuf, acc.at[dsts], ssem).wait()

    for b in range(2):
      load_src(b, b)
      issue_gather(b)

    def group_body(g, _):
      i0 = 2 * g
      for b in range(2):
        wait_gather(b)

        @pl.when(g > 0)
        def _():
          wait_scatter(b)
        scale_copy(b, i0 + b)
        load_dst(b, i0 + b)
        issue_scatter(b)

        @pl.when(g < n_groups - 1)
        def _():
          load_src(b, i0 + 2 + b)
          issue_gather(b)
      return 0
    lax.fori_loop(0, n_groups, group_body, 0)

    for b in range(2):
      wait_scatter(b)

    for r in range(rem):
      i = 2 * n_groups + r
      load_src(0, i)
      pltpu.sync_copy(table.at[bufs[0][0]], bufs[0][2])
      scale_copy(0, i)
      load_dst(0, i)
      pltpu.sync_copy(bufs[0][3], acc.at[bufs[0][1]], add=True)

    plsc.subcore_barrier()

    base_rows = rows_per_s // 8 * 8
    last_rows = n_nodes - (NS - 1) * base_rows
    for cc in range(NC):
      @pl.when((c == cc) & (s < NS - 1))
      def _():
        pltpu.sync_copy(acc.at[pl.ds(s * base_rows, base_rows)],
                        out.at[pl.ds(s * base_rows, base_rows),
                               pl.ds(cc * D, D)])

      @pl.when((c == cc) & (s == NS - 1))
      def _():
        pltpu.sync_copy(acc.at[pl.ds((NS - 1) * base_rows, last_rows)],
                        out.at[pl.ds((NS - 1) * base_rows, last_rows),
                               pl.ds(cc * D, D)])

  return agg


def _matmul_tc(n, f_in, f_out, bn):
  def body(x_ref, w_ref, o_ref):
    o_ref[...] = jnp.dot(x_ref[...], w_ref[...],
                         preferred_element_type=jnp.float32)
  return pl.pallas_call(
      body,
      grid=(n // bn,),
      in_specs=[
          pl.BlockSpec((bn, f_in), lambda i: (i, 0)),
          pl.BlockSpec((f_in, f_out), lambda i: (0, 0)),
      ],
      out_specs=pl.BlockSpec((bn, f_out), lambda i: (i, 0)),
      out_shape=jax.ShapeDtypeStruct((n, f_out), jnp.float32),
  )


def _relu_matmul_tc(n, f_in, f_out, bn):
  def body(p_ref, b_ref, w_ref, o_ref):
    h = jnp.maximum(p_ref[...] + b_ref[0], 0.0)
    o_ref[...] = jnp.dot(h, w_ref[...], preferred_element_type=jnp.float32)
  return pl.pallas_call(
      body,
      grid=(n // bn,),
      in_specs=[
          pl.BlockSpec((bn, f_in), lambda i: (i, 0)),
          pl.BlockSpec((1, f_in), lambda i: (0, 0)),
          pl.BlockSpec((f_in, f_out), lambda i: (0, 0)),
      ],
      out_specs=pl.BlockSpec((bn, f_out), lambda i: (i, 0)),
      out_shape=jax.ShapeDtypeStruct((n, f_out), jnp.float32),
  )


def _logsoftmax_tc(n, d, bn):
  def body(p_ref, b_ref, o_ref):
    p = p_ref[...]
    logits = p[:, :d] + p[:, d:] + b_ref[0]
    m = jnp.max(logits, axis=1, keepdims=True)
    shifted = logits - m
    lse = jnp.log(jnp.sum(jnp.exp(shifted), axis=1, keepdims=True))
    o_ref[...] = shifted - lse
  return pl.pallas_call(
      body,
      grid=(n // bn,),
      in_specs=[
          pl.BlockSpec((bn, 2 * d), lambda i: (i, 0)),
          pl.BlockSpec((1, d), lambda i: (0, 0)),
      ],
      out_specs=pl.BlockSpec((bn, d), lambda i: (i, 0)),
      out_shape=jax.ShapeDtypeStruct((n, d), jnp.float32),
  )


@jax.jit
def kernel(x, edge_index, adj_values, W1, b1, W2, b2):
  n, f_in = x.shape
  h_dim = W1.shape[1]
  o_dim = W2.shape[1]
  e = edge_index.shape[1]

  bn = 1000
  support1 = _matmul_tc(n, f_in, h_dim, bn)(x, W1)
  tbl1 = support1.reshape(NC * n, h_dim // NC)
  agg1 = _sc_agg(n, e, True)(tbl1, edge_index, adj_values)
  support2 = _relu_matmul_tc(n, h_dim, o_dim, bn)(
      agg1, b1.reshape(1, h_dim), W2)
  part2 = _sc_agg(n, e, False)(support2, edge_index, adj_values)
  return _logsoftmax_tc(n, o_dim, bn)(part2, b2.reshape(1, o_dim))


# speedup vs baseline: 1.2254x; 1.2237x over previous
import functools

import jax
import jax.numpy as jnp
from jax import lax
from jax.experimental import pallas as pl
from jax.experimental.pallas import tpu as pltpu
from jax.experimental.pallas import tpu_sc as plsc

NC = 2
NS = 16
NW = NC * NS
CHUNK = 80
ZROWS = 125
D = 64


def _sc_agg(n_nodes, n_edges, col_split):
  workers = NS if col_split else NW
  e_per = n_edges // workers
  n_chunks = -(-e_per // CHUNK)
  e_pad = n_chunks * CHUNK
  n_groups = n_chunks // 2
  rem = n_chunks - 2 * n_groups
  assert e_per % 16 == 0 and n_groups >= 2
  rows_per_s = n_nodes // NS
  nz = rows_per_s // ZROWS
  assert rows_per_s % ZROWS == 0
  nvec = D // 16
  n_tab = (NC * n_nodes) if col_split else n_nodes

  mesh = plsc.VectorSubcoreMesh(core_axis_name="c", subcore_axis_name="s")

  @functools.partial(
      pl.kernel,
      out_type=jax.ShapeDtypeStruct((n_nodes, NC * D), jnp.float32),
      mesh=mesh,
      compiler_params=pltpu.CompilerParams(use_tc_tiling_on_sc=False),
      scratch_types=[
          pltpu.VMEM((e_pad,), jnp.int32),
          pltpu.VMEM((e_pad,), jnp.int32),
          pltpu.VMEM((e_pad,), jnp.float32),
          *([pltpu.VMEM((CHUNK,), jnp.int32)] * 2),
          *([pltpu.VMEM((CHUNK,), jnp.int32)] * 2),
          *([pltpu.VMEM((CHUNK, D), jnp.float32)] * 2),
          *([pltpu.VMEM((CHUNK, D), jnp.float32)] * 2),
          pltpu.VMEM_SHARED((n_nodes, D), jnp.float32),
          *([pltpu.SemaphoreType.DMA] * 4),
      ],
  )
  def agg(table, edge, adj, out, src_v, dst_v, adj_v,
          srcs0, srcs1, dsts0, dsts1, grow0, grow1, srow0, srow1,
          acc, gsem0, gsem1, ssem0, ssem1):
    zbuf = srow0.at[pl.ds(0, ZROWS)]
    c = lax.axis_index("c")
    s = lax.axis_index("s")
    w = s if col_split else s * NC + c
    bufs = ((srcs0, dsts0, grow0, srow0, gsem0, ssem0),
            (srcs1, dsts1, grow1, srow1, gsem1, ssem1))

    def zrow(r, _):
      for j in range(nvec):
        zbuf[r, pl.ds(16 * j, 16)] = jnp.zeros((16,), jnp.float32)
      return 0
    lax.fori_loop(0, ZROWS, zrow, 0)
    for k in range(nz):
      pltpu.sync_copy(zbuf, acc.at[pl.ds(s * rows_per_s + k * ZROWS, ZROWS)])

    pltpu.sync_copy(edge.at[0, pl.ds(w * e_per, e_per)],
                    src_v.at[pl.ds(0, e_per)])
    pltpu.sync_copy(edge.at[1, pl.ds(w * e_per, e_per)],
                    dst_v.at[pl.ds(0, e_per)])
    pltpu.sync_copy(adj.at[pl.ds(w * e_per, e_per)],
                    adj_v.at[pl.ds(0, e_per)])
    for t in range((e_pad - e_per) // 16):
      src_v[pl.ds(e_per + 16 * t, 16)] = jnp.zeros((16,), jnp.int32)
      dst_v[pl.ds(e_per + 16 * t, 16)] = jnp.zeros((16,), jnp.int32)
      adj_v[pl.ds(e_per + 16 * t, 16)] = jnp.zeros((16,), jnp.float32)
    plsc.subcore_barrier()

    def load_src(b, i):
      srcs = bufs[b][0]
      for j in range(CHUNK // 16):
        v = src_v[pl.ds(i * CHUNK + 16 * j, 16)]
        if col_split:
          v = v * 2 + c
        srcs[pl.ds(16 * j, 16)] = v

    def load_dst(b, i):
      dsts = bufs[b][1]
      for j in range(CHUNK // 16):
        dsts[pl.ds(16 * j, 16)] = dst_v[pl.ds(i * CHUNK + 16 * j, 16)]

    def scale_copy(b, i):
      gbuf = bufs[b][2]
      sbuf = bufs[b][3]
      for g in range(CHUNK // 16):
        a16 = adj_v[pl.ds(i * CHUNK + 16 * g, 16)]
        for r in range(16):
          a = a16[r]
          for j in range(nvec):
            sl = pl.ds(16 * j, 16)
            sbuf[16 * g + r, sl] = gbuf[16 * g + r, sl] * a

    def issue_gather(b):
      srcs, _, gbuf, _, gsem, _ = bufs[b]
      pltpu.async_copy(table.at[srcs], gbuf, gsem)

    def wait_gather(b):
      srcs, _, gbuf, _, gsem, _ = bufs[b]
      pltpu.make_async_copy(table.at[srcs], gbuf, gsem).wait()

    def issue_scatter(b):
      _, dsts, _, sbuf, _, ssem = bufs[b]
      pltpu.async_copy(sbuf, acc.at[dsts], ssem, add=True)

    def wait_scatter(b):
      _, dsts, _, sbuf, _, ssem = bufs[b]
      pltpu.make_async_copy(sbuf, acc.at[dsts], ssem).wait()

    for b in range(2):
      load_src(b, b)
      issue_gather(b)

    def group_body(g, _):
      i0 = 2 * g
      for b in range(2):
        wait_gather(b)

        @pl.when(g > 0)
        def _():
          wait_scatter(b)
        scale_copy(b, i0 + b)
        load_dst(b, i0 + b)
        issue_scatter(b)

        @pl.when(g < n_groups - 1)
        def _():
          load_src(b, i0 + 2 + b)
          issue_gather(b)
      return 0
    lax.fori_loop(0, n_groups, group_body, 0)

    for b in range(2):
      wait_scatter(b)

    for r in range(rem):
      i = 2 * n_groups + r
      load_src(0, i)
      pltpu.sync_copy(table.at[bufs[0][0]], bufs[0][2])
      scale_copy(0, i)
      load_dst(0, i)
      pltpu.sync_copy(bufs[0][3], acc.at[bufs[0][1]], add=True)

    plsc.subcore_barrier()

    base_rows = rows_per_s // 8 * 8
    last_rows = n_nodes - (NS - 1) * base_rows
    for cc in range(NC):
      @pl.when((c == cc) & (s < NS - 1))
      def _():
        pltpu.sync_copy(acc.at[pl.ds(s * base_rows, base_rows)],
                        out.at[pl.ds(s * base_rows, base_rows),
                               pl.ds(cc * D, D)])

      @pl.when((c == cc) & (s == NS - 1))
      def _():
        pltpu.sync_copy(acc.at[pl.ds((NS - 1) * base_rows, last_rows)],
                        out.at[pl.ds((NS - 1) * base_rows, last_rows),
                               pl.ds(cc * D, D)])

  return agg


def _matmul_tc(n, f_in, f_out, bn):
  def body(x_ref, w_ref, o_ref):
    o_ref[...] = jnp.dot(x_ref[...], w_ref[...],
                         preferred_element_type=jnp.float32)
  return pl.pallas_call(
      body,
      grid=(n // bn,),
      in_specs=[
          pl.BlockSpec((bn, f_in), lambda i: (i, 0)),
          pl.BlockSpec((f_in, f_out), lambda i: (0, 0)),
      ],
      out_specs=pl.BlockSpec((bn, f_out), lambda i: (i, 0)),
      out_shape=jax.ShapeDtypeStruct((n, f_out), jnp.float32),
  )


def _relu_matmul_tc(n, f_in, f_out, bn):
  def body(p_ref, b_ref, w_ref, o_ref):
    h = jnp.maximum(p_ref[...] + b_ref[0], 0.0)
    o_ref[...] = jnp.dot(h, w_ref[...], preferred_element_type=jnp.float32)
  return pl.pallas_call(
      body,
      grid=(n // bn,),
      in_specs=[
          pl.BlockSpec((bn, f_in), lambda i: (i, 0)),
          pl.BlockSpec((1, f_in), lambda i: (0, 0)),
          pl.BlockSpec((f_in, f_out), lambda i: (0, 0)),
      ],
      out_specs=pl.BlockSpec((bn, f_out), lambda i: (i, 0)),
      out_shape=jax.ShapeDtypeStruct((n, f_out), jnp.float32),
  )


def _logsoftmax_tc(n, d, bn):
  def body(p_ref, b_ref, o_ref):
    p = p_ref[...]
    logits = p[:, :d] + p[:, d:] + b_ref[0]
    m = jnp.max(logits, axis=1, keepdims=True)
    shifted = logits - m
    lse = jnp.log(jnp.sum(jnp.exp(shifted), axis=1, keepdims=True))
    o_ref[...] = shifted - lse
  return pl.pallas_call(
      body,
      grid=(n // bn,),
      in_specs=[
          pl.BlockSpec((bn, 2 * d), lambda i: (i, 0)),
          pl.BlockSpec((1, d), lambda i: (0, 0)),
      ],
      out_specs=pl.BlockSpec((bn, d), lambda i: (i, 0)),
      out_shape=jax.ShapeDtypeStruct((n, d), jnp.float32),
  )


@jax.jit
def kernel(x, edge_index, adj_values, W1, b1, W2, b2):
  n, f_in = x.shape
  h_dim = W1.shape[1]
  o_dim = W2.shape[1]
  e = edge_index.shape[1]

  bn = 1000
  support1 = _matmul_tc(n, f_in, h_dim, bn)(x, W1)
  tbl1 = support1.reshape(NC * n, h_dim // NC)
  agg1 = _sc_agg(n, e, True)(tbl1, edge_index, adj_values)
  support2 = _relu_matmul_tc(n, h_dim, o_dim, bn)(
      agg1, b1.reshape(1, h_dim), W2)
  part2 = _sc_agg(n, e, False)(support2, edge_index, adj_values)
  return _logsoftmax_tc(n, o_dim, bn)(part2, b2.reshape(1, o_dim))


# speedup vs baseline: 1.2543x; 1.0236x over previous
import functools

import jax
import jax.numpy as jnp
from jax import lax
from jax.experimental import pallas as pl
from jax.experimental.pallas import tpu as pltpu
from jax.experimental.pallas import tpu_sc as plsc

NC = 2
NS = 16
NW = NC * NS
CHUNK = 80
ZROWS = 125
D = 64


def _sc_agg(n_nodes, n_edges, col_split):
  workers = NS if col_split else NW
  e_per = n_edges // workers
  n_chunks = -(-e_per // CHUNK)
  e_pad = n_chunks * CHUNK
  n_groups = n_chunks // 2
  rem = n_chunks - 2 * n_groups
  assert e_per % 16 == 0 and n_groups >= 2
  rows_per_s = n_nodes // NS
  nz = rows_per_s // ZROWS
  assert rows_per_s % ZROWS == 0
  nvec = D // 16
  n_tab = (NC * n_nodes) if col_split else n_nodes

  mesh = plsc.VectorSubcoreMesh(core_axis_name="c", subcore_axis_name="s")

  @functools.partial(
      pl.kernel,
      out_type=jax.ShapeDtypeStruct((n_nodes, NC * D), jnp.float32),
      mesh=mesh,
      compiler_params=pltpu.CompilerParams(use_tc_tiling_on_sc=False),
      scratch_types=[
          pltpu.VMEM((e_pad,), jnp.int32),
          pltpu.VMEM((e_pad,), jnp.int32),
          pltpu.VMEM((e_pad,), jnp.float32),
          *([pltpu.VMEM((CHUNK,), jnp.int32)] * 2),
          *([pltpu.VMEM((CHUNK,), jnp.int32)] * 2),
          *([pltpu.VMEM((CHUNK, D), jnp.float32)] * 2),
          *([pltpu.VMEM((CHUNK, D), jnp.float32)] * 2),
          pltpu.VMEM_SHARED((n_nodes, D), jnp.float32),
          *([pltpu.SemaphoreType.DMA] * 4),
      ],
  )
  def agg(table, edge, adj, out, src_v, dst_v, adj_v,
          srcs0, srcs1, dsts0, dsts1, grow0, grow1, srow0, srow1,
          acc, gsem0, gsem1, ssem0, ssem1):
    zbuf = srow0.at[pl.ds(0, ZROWS)]
    c = lax.axis_index("c")
    s = lax.axis_index("s")
    w = s if col_split else s * NC + c
    bufs = ((srcs0, dsts0, grow0, srow0, gsem0, ssem0),
            (srcs1, dsts1, grow1, srow1, gsem1, ssem1))

    def zrow(r, _):
      for j in range(nvec):
        zbuf[r, pl.ds(16 * j, 16)] = jnp.zeros((16,), jnp.float32)
      return 0
    lax.fori_loop(0, ZROWS, zrow, 0)
    for k in range(nz):
      pltpu.sync_copy(zbuf, acc.at[pl.ds(s * rows_per_s + k * ZROWS, ZROWS)])

    pltpu.sync_copy(edge.at[0, pl.ds(w * e_per, e_per)],
                    src_v.at[pl.ds(0, e_per)])
    pltpu.sync_copy(edge.at[1, pl.ds(w * e_per, e_per)],
                    dst_v.at[pl.ds(0, e_per)])
    pltpu.sync_copy(adj.at[pl.ds(w * e_per, e_per)],
                    adj_v.at[pl.ds(0, e_per)])
    for t in range((e_pad - e_per) // 16):
      src_v[pl.ds(e_per + 16 * t, 16)] = jnp.zeros((16,), jnp.int32)
      dst_v[pl.ds(e_per + 16 * t, 16)] = jnp.zeros((16,), jnp.int32)
      adj_v[pl.ds(e_per + 16 * t, 16)] = jnp.zeros((16,), jnp.float32)
    plsc.subcore_barrier()

    def load_src(b, i):
      srcs = bufs[b][0]
      for j in range(CHUNK // 16):
        v = src_v[pl.ds(i * CHUNK + 16 * j, 16)]
        if col_split:
          v = v * 2 + c
        srcs[pl.ds(16 * j, 16)] = v

    def load_dst(b, i):
      dsts = bufs[b][1]
      for j in range(CHUNK // 16):
        dsts[pl.ds(16 * j, 16)] = dst_v[pl.ds(i * CHUNK + 16 * j, 16)]

    def scale_copy(b, i):
      gbuf = bufs[b][2]
      sbuf = bufs[b][3]
      for g in range(CHUNK // 16):
        a16 = adj_v[pl.ds(i * CHUNK + 16 * g, 16)]
        for r in range(16):
          a = a16[r]
          for j in range(nvec):
            sl = pl.ds(16 * j, 16)
            sbuf[16 * g + r, sl] = gbuf[16 * g + r, sl] * a

    def issue_gather(b):
      srcs, _, gbuf, _, gsem, _ = bufs[b]
      pltpu.async_copy(table.at[srcs], gbuf, gsem)

    def wait_gather(b):
      srcs, _, gbuf, _, gsem, _ = bufs[b]
      pltpu.make_async_copy(table.at[srcs], gbuf, gsem).wait()

    def issue_scatter(b):
      _, dsts, _, sbuf, _, ssem = bufs[b]
      pltpu.async_copy(sbuf, acc.at[dsts], ssem, add=True)

    def wait_scatter(b):
      _, dsts, _, sbuf, _, ssem = bufs[b]
      pltpu.make_async_copy(sbuf, acc.at[dsts], ssem).wait()

    for b in range(2):
      load_src(b, b)
      issue_gather(b)

    def group_body(g, _):
      i0 = 2 * g
      for b in range(2):
        wait_gather(b)

        @pl.when(g > 0)
        def _():
          wait_scatter(b)
        scale_copy(b, i0 + b)
        load_dst(b, i0 + b)
        issue_scatter(b)

        @pl.when(g < n_groups - 1)
        def _():
          load_src(b, i0 + 2 + b)
          issue_gather(b)
      return 0
    lax.fori_loop(0, n_groups, group_body, 0)

    for b in range(2):
      wait_scatter(b)

    for r in range(rem):
      i = 2 * n_groups + r
      load_src(0, i)
      pltpu.sync_copy(table.at[bufs[0][0]], bufs[0][2])
      scale_copy(0, i)
      load_dst(0, i)
      pltpu.sync_copy(bufs[0][3], acc.at[bufs[0][1]], add=True)

    plsc.subcore_barrier()

    base_rows = rows_per_s // 8 * 8
    last_rows = n_nodes - (NS - 1) * base_rows
    for cc in range(NC):
      @pl.when((c == cc) & (s < NS - 1))
      def _():
        pltpu.sync_copy(acc.at[pl.ds(s * base_rows, base_rows)],
                        out.at[pl.ds(s * base_rows, base_rows),
                               pl.ds(cc * D, D)])

      @pl.when((c == cc) & (s == NS - 1))
      def _():
        pltpu.sync_copy(acc.at[pl.ds((NS - 1) * base_rows, last_rows)],
                        out.at[pl.ds((NS - 1) * base_rows, last_rows),
                               pl.ds(cc * D, D)])

  return agg


def _matmul_tc(n, f_in, f_out, bn):
  def body(x_ref, w_ref, o_ref):
    o_ref[...] = jnp.dot(x_ref[...], w_ref[...],
                         preferred_element_type=jnp.float32)
  return pl.pallas_call(
      body,
      grid=(n // bn,),
      in_specs=[
          pl.BlockSpec((bn, f_in), lambda i: (i, 0)),
          pl.BlockSpec((f_in, f_out), lambda i: (0, 0)),
      ],
      out_specs=pl.BlockSpec((bn, f_out), lambda i: (i, 0)),
      out_shape=jax.ShapeDtypeStruct((n, f_out), jnp.float32),
  )


def _relu_matmul_tc(n, f_in, f_out, bn):
  def body(p_ref, b_ref, w_ref, o_ref):
    h = jnp.maximum(p_ref[...] + b_ref[0], 0.0)
    o_ref[...] = jnp.dot(h, w_ref[...], preferred_element_type=jnp.float32)
  return pl.pallas_call(
      body,
      grid=(n // bn,),
      in_specs=[
          pl.BlockSpec((bn, f_in), lambda i: (i, 0)),
          pl.BlockSpec((1, f_in), lambda i: (0, 0)),
          pl.BlockSpec((f_in, f_out), lambda i: (0, 0)),
      ],
      out_specs=pl.BlockSpec((bn, f_out), lambda i: (i, 0)),
      out_shape=jax.ShapeDtypeStruct((n, f_out), jnp.float32),
  )


def _logsoftmax_tc(n, d, bn):
  def body(p_ref, b_ref, o_ref):
    p = p_ref[...]
    logits = p[:, :d] + p[:, d:] + b_ref[0]
    m = jnp.max(logits, axis=1, keepdims=True)
    shifted = logits - m
    lse = jnp.log(jnp.sum(jnp.exp(shifted), axis=1, keepdims=True))
    o_ref[...] = shifted - lse
  return pl.pallas_call(
      body,
      grid=(n // bn,),
      in_specs=[
          pl.BlockSpec((bn, 2 * d), lambda i: (i, 0)),
          pl.BlockSpec((1, d), lambda i: (0, 0)),
      ],
      out_specs=pl.BlockSpec((bn, d), lambda i: (i, 0)),
      out_shape=jax.ShapeDtypeStruct((n, d), jnp.float32),
  )


@jax.jit
def kernel(x, edge_index, adj_values, W1, b1, W2, b2):
  n, f_in = x.shape
  h_dim = W1.shape[1]
  o_dim = W2.shape[1]
  e = edge_index.shape[1]

  bn = 2000
  support1 = _matmul_tc(n, f_in, h_dim, bn)(x, W1)
  tbl1 = support1.reshape(NC * n, h_dim // NC)
  agg1 = _sc_agg(n, e, True)(tbl1, edge_index, adj_values)
  support2 = _relu_matmul_tc(n, h_dim, o_dim, bn)(
      agg1, b1.reshape(1, h_dim), W2)
  part2 = _sc_agg(n, e, False)(support2, edge_index, adj_values)
  return _logsoftmax_tc(n, o_dim, bn)(part2, b2.reshape(1, o_dim))


# speedup vs baseline: 1.2779x; 1.0188x over previous
import functools

import jax
import jax.numpy as jnp
from jax import lax
from jax.experimental import pallas as pl
from jax.experimental.pallas import tpu as pltpu
from jax.experimental.pallas import tpu_sc as plsc

NC = 2
NS = 16
NW = NC * NS
CHUNK = 80
ZROWS = 125
D = 64


def _sc_agg(n_nodes, n_edges, col_split):
  workers = NS if col_split else NW
  e_per = n_edges // workers
  n_chunks = -(-e_per // CHUNK)
  e_pad = n_chunks * CHUNK
  n_groups = n_chunks // 2
  rem = n_chunks - 2 * n_groups
  assert e_per % 16 == 0 and n_groups >= 2
  rows_per_s = n_nodes // NS
  nz = rows_per_s // ZROWS
  assert rows_per_s % ZROWS == 0
  nvec = D // 16
  n_tab = (NC * n_nodes) if col_split else n_nodes

  mesh = plsc.VectorSubcoreMesh(core_axis_name="c", subcore_axis_name="s")

  @functools.partial(
      pl.kernel,
      out_type=jax.ShapeDtypeStruct((n_nodes, NC * D), jnp.float32),
      mesh=mesh,
      compiler_params=pltpu.CompilerParams(use_tc_tiling_on_sc=False),
      scratch_types=[
          pltpu.VMEM((e_pad,), jnp.int32),
          pltpu.VMEM((e_pad,), jnp.int32),
          pltpu.VMEM((e_pad,), jnp.float32),
          *([pltpu.VMEM((CHUNK,), jnp.int32)] * 2),
          *([pltpu.VMEM((CHUNK,), jnp.int32)] * 2),
          *([pltpu.VMEM((CHUNK, D), jnp.float32)] * 2),
          *([pltpu.VMEM((CHUNK, D), jnp.float32)] * 2),
          pltpu.VMEM_SHARED((n_nodes, D), jnp.float32),
          *([pltpu.SemaphoreType.DMA] * 4),
      ],
  )
  def agg(table, edge, adj, out, src_v, dst_v, adj_v,
          srcs0, srcs1, dsts0, dsts1, grow0, grow1, srow0, srow1,
          acc, gsem0, gsem1, ssem0, ssem1):
    zbuf = srow0.at[pl.ds(0, ZROWS)]
    c = lax.axis_index("c")
    s = lax.axis_index("s")
    w = s if col_split else s * NC + c
    bufs = ((srcs0, dsts0, grow0, srow0, gsem0, ssem0),
            (srcs1, dsts1, grow1, srow1, gsem1, ssem1))

    def zrow(r, _):
      for j in range(nvec):
        zbuf[r, pl.ds(16 * j, 16)] = jnp.zeros((16,), jnp.float32)
      return 0
    lax.fori_loop(0, ZROWS, zrow, 0)
    for k in range(nz):
      pltpu.sync_copy(zbuf, acc.at[pl.ds(s * rows_per_s + k * ZROWS, ZROWS)])

    pltpu.sync_copy(edge.at[0, pl.ds(w * e_per, e_per)],
                    src_v.at[pl.ds(0, e_per)])
    pltpu.sync_copy(edge.at[1, pl.ds(w * e_per, e_per)],
                    dst_v.at[pl.ds(0, e_per)])
    pltpu.sync_copy(adj.at[pl.ds(w * e_per, e_per)],
                    adj_v.at[pl.ds(0, e_per)])
    for t in range((e_pad - e_per) // 16):
      src_v[pl.ds(e_per + 16 * t, 16)] = jnp.zeros((16,), jnp.int32)
      dst_v[pl.ds(e_per + 16 * t, 16)] = jnp.zeros((16,), jnp.int32)
      adj_v[pl.ds(e_per + 16 * t, 16)] = jnp.zeros((16,), jnp.float32)
    plsc.subcore_barrier()

    def load_src(b, i):
      srcs = bufs[b][0]
      for j in range(CHUNK // 16):
        v = src_v[pl.ds(i * CHUNK + 16 * j, 16)]
        if col_split:
          v = v * 2 + c
        srcs[pl.ds(16 * j, 16)] = v

    def load_dst(b, i):
      dsts = bufs[b][1]
      for j in range(CHUNK // 16):
        dsts[pl.ds(16 * j, 16)] = dst_v[pl.ds(i * CHUNK + 16 * j, 16)]

    def scale_copy(b, i):
      gbuf = bufs[b][2]
      sbuf = bufs[b][3]
      for g in range(CHUNK // 16):
        a16 = adj_v[pl.ds(i * CHUNK + 16 * g, 16)]
        for r in range(16):
          a = a16[r]
          for j in range(nvec):
            sl = pl.ds(16 * j, 16)
            sbuf[16 * g + r, sl] = gbuf[16 * g + r, sl] * a

    def issue_gather(b):
      srcs, _, gbuf, _, gsem, _ = bufs[b]
      pltpu.async_copy(table.at[srcs], gbuf, gsem)

    def wait_gather(b):
      srcs, _, gbuf, _, gsem, _ = bufs[b]
      pltpu.make_async_copy(table.at[srcs], gbuf, gsem).wait()

    def issue_scatter(b):
      _, dsts, _, sbuf, _, ssem = bufs[b]
      pltpu.async_copy(sbuf, acc.at[dsts], ssem, add=True)

    def wait_scatter(b):
      _, dsts, _, sbuf, _, ssem = bufs[b]
      pltpu.make_async_copy(sbuf, acc.at[dsts], ssem).wait()

    for b in range(2):
      load_src(b, b)
      issue_gather(b)

    def group_body(g, _):
      i0 = 2 * g
      for b in range(2):
        wait_gather(b)

        @pl.when(g > 0)
        def _():
          wait_scatter(b)
        scale_copy(b, i0 + b)
        load_dst(b, i0 + b)
        issue_scatter(b)

        @pl.when(g < n_groups - 1)
        def _():
          load_src(b, i0 + 2 + b)
          issue_gather(b)
      return 0
    lax.fori_loop(0, n_groups, group_body, 0)

    for b in range(2):
      wait_scatter(b)

    for r in range(rem):
      i = 2 * n_groups + r
      load_src(0, i)
      pltpu.sync_copy(table.at[bufs[0][0]], bufs[0][2])
      scale_copy(0, i)
      load_dst(0, i)
      pltpu.sync_copy(bufs[0][3], acc.at[bufs[0][1]], add=True)

    plsc.subcore_barrier()

    base_rows = rows_per_s // 8 * 8
    last_rows = n_nodes - (NS - 1) * base_rows
    for cc in range(NC):
      @pl.when((c == cc) & (s < NS - 1))
      def _():
        pltpu.sync_copy(acc.at[pl.ds(s * base_rows, base_rows)],
                        out.at[pl.ds(s * base_rows, base_rows),
                               pl.ds(cc * D, D)])

      @pl.when((c == cc) & (s == NS - 1))
      def _():
        pltpu.sync_copy(acc.at[pl.ds((NS - 1) * base_rows, last_rows)],
                        out.at[pl.ds((NS - 1) * base_rows, last_rows),
                               pl.ds(cc * D, D)])

  return agg


def _matmul_tc(n, f_in, f_out, bn):
  def body(x_ref, w_ref, o_ref):
    o_ref[...] = jnp.dot(x_ref[...], w_ref[...],
                         preferred_element_type=jnp.float32)
  return pl.pallas_call(
      body,
      grid=(n // bn,),
      in_specs=[
          pl.BlockSpec((bn, f_in), lambda i: (i, 0)),
          pl.BlockSpec((f_in, f_out), lambda i: (0, 0)),
      ],
      out_specs=pl.BlockSpec((bn, f_out), lambda i: (i, 0)),
      out_shape=jax.ShapeDtypeStruct((n, f_out), jnp.float32),
  )


def _relu_matmul_tc(n, f_in, f_out, bn):
  def body(p_ref, b_ref, w_ref, o_ref):
    h = jnp.maximum(p_ref[...] + b_ref[0], 0.0)
    o_ref[...] = jnp.dot(h, w_ref[...], preferred_element_type=jnp.float32)
  return pl.pallas_call(
      body,
      grid=(n // bn,),
      in_specs=[
          pl.BlockSpec((bn, f_in), lambda i: (i, 0)),
          pl.BlockSpec((1, f_in), lambda i: (0, 0)),
          pl.BlockSpec((f_in, f_out), lambda i: (0, 0)),
      ],
      out_specs=pl.BlockSpec((bn, f_out), lambda i: (i, 0)),
      out_shape=jax.ShapeDtypeStruct((n, f_out), jnp.float32),
  )


def _logsoftmax_tc(n, d, bn):
  def body(p_ref, b_ref, o_ref):
    p = p_ref[...]
    logits = p[:, :d] + p[:, d:] + b_ref[0]
    m = jnp.max(logits, axis=1, keepdims=True)
    shifted = logits - m
    lse = jnp.log(jnp.sum(jnp.exp(shifted), axis=1, keepdims=True))
    o_ref[...] = shifted - lse
  return pl.pallas_call(
      body,
      grid=(n // bn,),
      in_specs=[
          pl.BlockSpec((bn, 2 * d), lambda i: (i, 0)),
          pl.BlockSpec((1, d), lambda i: (0, 0)),
      ],
      out_specs=pl.BlockSpec((bn, d), lambda i: (i, 0)),
      out_shape=jax.ShapeDtypeStruct((n, d), jnp.float32),
  )


@jax.jit
def kernel(x, edge_index, adj_values, W1, b1, W2, b2):
  n, f_in = x.shape
  h_dim = W1.shape[1]
  o_dim = W2.shape[1]
  e = edge_index.shape[1]

  bn = 5000
  support1 = _matmul_tc(n, f_in, h_dim, bn)(x, W1)
  tbl1 = support1.reshape(NC * n, h_dim // NC)
  agg1 = _sc_agg(n, e, True)(tbl1, edge_index, adj_values)
  support2 = _relu_matmul_tc(n, h_dim, o_dim, bn)(
      agg1, b1.reshape(1, h_dim), W2)
  part2 = _sc_agg(n, e, False)(support2, edge_index, adj_values)
  return _logsoftmax_tc(n, o_dim, bn)(part2, b2.reshape(1, o_dim))
